# baseline (speedup 1.0000x reference)
import functools

import jax
import jax.numpy as jnp
from jax import lax
from jax.experimental import pallas as pl
from jax.experimental.pallas import tpu as pltpu
from jax.experimental.pallas import tpu_sc as plsc

_PATCH = 16
_EMBED = 768
_HW = 512
_HP = _HW // _PATCH
_NPATCH = _HP * _HP
_NWIN = 4
_CH = 32
_NCH = _NPATCH // _CH
_NBUF = 4

_ROWS = (
    (145, 146, 147, 148, 149, 150, 151, 177, 178, 179, 180, 181, 182, 183,
     209, 210, 211, 212, 213, 214, 215, 241, 242, 243, 244, 245, 246, 247,
     273, 274, 275, 276, 277, 278, 279, 305, 306, 307, 308),
    (755, 756, 757, 758, 759, 760, 761, 787, 788, 789, 790, 791, 792, 793,
     819, 820, 821, 822, 823, 824, 825, 851, 852, 853, 854, 855, 856, 857,
     883, 884, 885, 886, 887, 888, 889, 915, 916, 917, 918),
    (588, 589, 590, 591, 592, 593, 594, 620, 621, 622, 623, 624, 625, 626,
     652, 653, 654, 655, 656, 657, 658, 684, 685, 686, 687, 688, 689, 690,
     716, 717, 718, 719, 720, 721, 722, 748, 749, 750, 751),
    (41, 42, 43, 44, 45, 46, 47, 73, 74, 75, 76, 77, 78, 79,
     105, 106, 107, 108, 109, 110, 111, 137, 138, 139, 140, 141, 142, 143,
     169, 170, 171, 172, 173, 174, 175, 201, 202, 203, 204),
)


def _runs(rows):
    out = []
    for r in rows:
        if out and out[-1][0] + out[-1][1] == r:
            out[-1] = (out[-1][0], out[-1][1] + 1)
        else:
            out.append((r, 1))
    return tuple(out)


_ROW_RUNS = tuple(_runs(sorted(rows)) for rows in _ROWS)
_MAXRUN = max(l for runs in _ROW_RUNS for _, l in runs)


def _tc_embed(xp_ref, wt_ref, b_ref, y_ref):
    y_ref[0] = (jnp.dot(xp_ref[0], wt_ref[...],
                        preferred_element_type=jnp.float32) + b_ref[...])


_CHW = _CH * _EMBED


def _sc_expand(y_hbm, mt_hbm, out_hbm, buf0, buf1, buf2, buf3, mtbuf,
               si0, si1, si2, si3, so0, so1, so2, so3, sm):
    wid = lax.axis_index("s") * 2 + lax.axis_index("c")
    b = wid // _NWIN
    w = wid % _NWIN
    y_base = b * (_NPATCH * _EMBED)
    o_base = (b * _NWIN + w) * (_NPATCH * _EMBED)

    for j in range(_MAXRUN):
        pltpu.sync_copy(mt_hbm, mtbuf.at[pl.ds(j * _EMBED, _EMBED)])

    bufs = (buf0, buf1, buf2, buf3)
    sis = (si0, si1, si2, si3)
    sos = (so0, so1, so2, so3)

    def start_in(k):
        return pltpu.async_copy(
            y_hbm.at[pl.ds(y_base + k * _CHW, _CHW)], bufs[k % _NBUF],
            sis[k % _NBUF])

    hin = {k: start_in(k) for k in range(_NBUF)}
    hout = {}
    unwaited = set()
    for k in range(_NCH):
        hin[k].wait()
        hout[k] = pltpu.async_copy(
            bufs[k % _NBUF], out_hbm.at[pl.ds(o_base + k * _CHW, _CHW)],
            sos[k % _NBUF])
        unwaited.add(k)
        j = k - (_NBUF - 1)
        if j >= 0 and j + _NBUF < _NCH:
            hout[j].wait()
            unwaited.discard(j)
            hin[j + _NBUF] = start_in(j + _NBUF)
    for k in sorted(unwaited):
        hout[k].wait()

    for ww in range(_NWIN):
        @pl.when(w == ww)
        def _(ww=ww):
            hs = [pltpu.async_copy(
                      mtbuf.at[pl.ds(0, length * _EMBED)],
                      out_hbm.at[pl.ds(o_base + start * _EMBED,
                                       length * _EMBED)], sm)
                  for start, length in _ROW_RUNS[ww]]
            for h in hs:
                h.wait()


def kernel(x, W, b, mask_token):
    Bn = x.shape[0]
    xp = x.reshape(Bn, 3, _HP, _PATCH, _HP, _PATCH)
    xp = xp.transpose(0, 2, 4, 1, 3, 5).reshape(Bn, _NPATCH, 3 * _PATCH * _PATCH)
    xp = xp.astype(jnp.bfloat16)
    wt = W.reshape(_EMBED, 3 * _PATCH * _PATCH).T.astype(jnp.bfloat16)
    b2 = b.reshape(1, _EMBED)
    mt2 = mask_token.reshape(1, _EMBED)

    y = pl.pallas_call(
        _tc_embed,
        grid=(Bn,),
        in_specs=[
            pl.BlockSpec((1, _NPATCH, 3 * _PATCH * _PATCH),
                         lambda i: (i, 0, 0)),
            pl.BlockSpec((3 * _PATCH * _PATCH, _EMBED), lambda i: (0, 0)),
            pl.BlockSpec((1, _EMBED), lambda i: (0, 0)),
        ],
        out_specs=pl.BlockSpec((1, _NPATCH, _EMBED), lambda i: (i, 0, 0)),
        out_shape=jax.ShapeDtypeStruct((Bn, _NPATCH, _EMBED), jnp.float32),
        compiler_params=pltpu.CompilerParams(
            dimension_semantics=("parallel",)),
    )(xp, wt, b2)

    sc_expand = pl.kernel(
        _sc_expand,
        out_type=jax.ShapeDtypeStruct((Bn * _NWIN * _NPATCH * _EMBED,),
                                      jnp.float32),
        mesh=plsc.VectorSubcoreMesh(core_axis_name="c",
                                    subcore_axis_name="s"),
        scratch_types=(
            [pltpu.VMEM((_CHW,), jnp.float32) for _ in range(_NBUF)]
            + [pltpu.VMEM((_MAXRUN * _EMBED,), jnp.float32)]
            + [pltpu.SemaphoreType.DMA for _ in range(2 * _NBUF + 1)]
        ),
    )
    out_flat = sc_expand(y.reshape(-1), mt2.reshape(-1))
    return out_flat.reshape(Bn, _NWIN, _NPATCH, _EMBED)


# speedup vs baseline: 1.7114x; 1.7114x over previous
import functools
import math

import jax
import jax.numpy as jnp
import numpy as np
from jax.experimental import pallas as pl
from jax.experimental.pallas import tpu as pltpu

_PATCH = 16
_EMBED = 768
_HW = 512
_HP = _HW // _PATCH
_NPATCH = _HP * _HP
_WIN = 7
_NWIN = 4
_RATIO = 0.8
_EBLK = 768


_ROWS = (
    (145, 146, 147, 148, 149, 150, 151, 177, 178, 179, 180, 181, 182, 183,
     209, 210, 211, 212, 213, 214, 215, 241, 242, 243, 244, 245, 246, 247,
     273, 274, 275, 276, 277, 278, 279, 305, 306, 307, 308),
    (755, 756, 757, 758, 759, 760, 761, 787, 788, 789, 790, 791, 792, 793,
     819, 820, 821, 822, 823, 824, 825, 851, 852, 853, 854, 855, 856, 857,
     883, 884, 885, 886, 887, 888, 889, 915, 916, 917, 918),
    (588, 589, 590, 591, 592, 593, 594, 620, 621, 622, 623, 624, 625, 626,
     652, 653, 654, 655, 656, 657, 658, 684, 685, 686, 687, 688, 689, 690,
     716, 717, 718, 719, 720, 721, 722, 748, 749, 750, 751),
    (41, 42, 43, 44, 45, 46, 47, 73, 74, 75, 76, 77, 78, 79,
     105, 106, 107, 108, 109, 110, 111, 137, 138, 139, 140, 141, 142, 143,
     169, 170, 171, 172, 173, 174, 175, 201, 202, 203, 204),
)


def _runs(rows):
    out = []
    for r in rows:
        if out and out[-1][0] + out[-1][1] == r:
            out[-1] = (out[-1][0], out[-1][1] + 1)
        else:
            out.append((r, 1))
    return tuple(out)


_ROW_RUNS = tuple(_runs(sorted(rows)) for rows in _ROWS)


def _mae_kernel(runs, xp_ref, wt_ref, b_ref, mt_ref, out_ref):
    y = jnp.dot(xp_ref[0], wt_ref[...], preferred_element_type=jnp.float32)
    y = y + b_ref[...]
    for w in range(_NWIN):
        out_ref[0, w] = y
    for w in range(_NWIN):
        for start, length in runs[w]:
            out_ref[0, w, pl.ds(start, length)] = jnp.broadcast_to(
                mt_ref[...], (length, mt_ref.shape[1]))


def kernel(x, W, b, mask_token):
    Bn = x.shape[0]
    xp = x.reshape(Bn, 3, _HP, _PATCH, _HP, _PATCH)
    xp = xp.transpose(0, 2, 4, 1, 3, 5).reshape(Bn, _NPATCH, 3 * _PATCH * _PATCH)
    xp = xp.astype(jnp.bfloat16)
    wt = W.reshape(_EMBED, 3 * _PATCH * _PATCH).T.astype(jnp.bfloat16)
    b2 = b.reshape(1, _EMBED)
    mt2 = mask_token.reshape(1, _EMBED)

    n_eblk = _EMBED // _EBLK
    out = pl.pallas_call(
        functools.partial(_mae_kernel, _ROW_RUNS),
        grid=(Bn, n_eblk),
        in_specs=[
            pl.BlockSpec((1, _NPATCH, 3 * _PATCH * _PATCH),
                         lambda i, e: (i, 0, 0)),
            pl.BlockSpec((3 * _PATCH * _PATCH, _EBLK), lambda i, e: (0, e)),
            pl.BlockSpec((1, _EBLK), lambda i, e: (0, e)),
            pl.BlockSpec((1, _EBLK), lambda i, e: (0, e)),
        ],
        out_specs=pl.BlockSpec((1, _NWIN, _NPATCH, _EBLK),
                               lambda i, e: (i, 0, 0, e)),
        out_shape=jax.ShapeDtypeStruct((Bn, _NWIN, _NPATCH, _EMBED),
                                       jnp.float32),
        compiler_params=pltpu.CompilerParams(
            dimension_semantics=("parallel", "parallel")),
    )(xp, wt, b2, mt2)
    return out


# speedup vs baseline: 1.7123x; 1.0005x over previous
import functools
import math

import jax
import jax.numpy as jnp
import numpy as np
from jax.experimental import pallas as pl
from jax.experimental.pallas import tpu as pltpu

_PATCH = 16
_EMBED = 768
_HW = 512
_HP = _HW // _PATCH
_NPATCH = _HP * _HP
_WIN = 7
_NWIN = 4
_RATIO = 0.8
_EBLK = 768


_ROWS = (
    (145, 146, 147, 148, 149, 150, 151, 177, 178, 179, 180, 181, 182, 183,
     209, 210, 211, 212, 213, 214, 215, 241, 242, 243, 244, 245, 246, 247,
     273, 274, 275, 276, 277, 278, 279, 305, 306, 307, 308),
    (755, 756, 757, 758, 759, 760, 761, 787, 788, 789, 790, 791, 792, 793,
     819, 820, 821, 822, 823, 824, 825, 851, 852, 853, 854, 855, 856, 857,
     883, 884, 885, 886, 887, 888, 889, 915, 916, 917, 918),
    (588, 589, 590, 591, 592, 593, 594, 620, 621, 622, 623, 624, 625, 626,
     652, 653, 654, 655, 656, 657, 658, 684, 685, 686, 687, 688, 689, 690,
     716, 717, 718, 719, 720, 721, 722, 748, 749, 750, 751),
    (41, 42, 43, 44, 45, 46, 47, 73, 74, 75, 76, 77, 78, 79,
     105, 106, 107, 108, 109, 110, 111, 137, 138, 139, 140, 141, 142, 143,
     169, 170, 171, 172, 173, 174, 175, 201, 202, 203, 204),
)


def _runs(rows):
    out = []
    for r in rows:
        if out and out[-1][0] + out[-1][1] == r:
            out[-1] = (out[-1][0], out[-1][1] + 1)
        else:
            out.append((r, 1))
    return tuple(out)


_ROW_RUNS = tuple(_runs(sorted(rows)) for rows in _ROWS)


def _mae_kernel(runs, xp_ref, wt_ref, b_ref, mt_ref, out_ref):
    y = jnp.dot(xp_ref[0], wt_ref[...], preferred_element_type=jnp.float32)
    y = y + b_ref[...]
    for w in range(_NWIN):
        out_ref[0, w] = y
    for w in range(_NWIN):
        for start, length in runs[w]:
            out_ref[0, w, pl.ds(start, length)] = jnp.broadcast_to(
                mt_ref[...], (length, mt_ref.shape[1]))


def kernel(x, W, b, mask_token):
    Bn = x.shape[0]
    xp = x.astype(jnp.bfloat16).reshape(Bn, 3, _HP, _PATCH, _HP, _PATCH)
    xp = xp.transpose(0, 2, 4, 1, 3, 5).reshape(Bn, _NPATCH, 3 * _PATCH * _PATCH)
    wt = W.reshape(_EMBED, 3 * _PATCH * _PATCH).T.astype(jnp.bfloat16)
    b2 = b.reshape(1, _EMBED)
    mt2 = mask_token.reshape(1, _EMBED)

    n_eblk = _EMBED // _EBLK
    out = pl.pallas_call(
        functools.partial(_mae_kernel, _ROW_RUNS),
        grid=(Bn, n_eblk),
        in_specs=[
            pl.BlockSpec((1, _NPATCH, 3 * _PATCH * _PATCH),
                         lambda i, e: (i, 0, 0)),
            pl.BlockSpec((3 * _PATCH * _PATCH, _EBLK), lambda i, e: (0, e)),
            pl.BlockSpec((1, _EBLK), lambda i, e: (0, e)),
            pl.BlockSpec((1, _EBLK), lambda i, e: (0, e)),
        ],
        out_specs=pl.BlockSpec((1, _NWIN, _NPATCH, _EBLK),
                               lambda i, e: (i, 0, 0, e)),
        out_shape=jax.ShapeDtypeStruct((Bn, _NWIN, _NPATCH, _EMBED),
                                       jnp.float32),
        compiler_params=pltpu.CompilerParams(
            dimension_semantics=("parallel", "parallel")),
    )(xp, wt, b2, mt2)
    return out
